# baseline (speedup 1.0000x reference)
import functools

import jax
import jax.numpy as jnp
from jax.experimental import pallas as pl
from jax.experimental.pallas import tpu as pltpu

B, N, D, E, H, K = 1, 2048, 768, 8, 8, 2
HD = D // H
R = 512
NB = N // R


def _stage1_body(x_ref, inw_ref, inb_ref, rw_ref,
                 xn_ref, idx_ref, tkw_ref, aux_ref):
    x = x_ref[...]
    rm = jnp.mean(x, axis=0, keepdims=True)
    logits = jax.lax.dot_general(
        rm, rw_ref[...], (((1,), (1,)), ((), ())),
        preferred_element_type=jnp.float32)
    lmax = jnp.max(logits)
    ex = jnp.exp(logits - lmax)
    probs = ex / jnp.sum(ex)
    iota = jax.lax.broadcasted_iota(jnp.int32, (1, E), 1)
    m1 = jnp.max(probs)
    i1 = jnp.min(jnp.where(probs == m1, iota, E))
    masked = jnp.where(iota == i1, -jnp.inf, probs)
    m2 = jnp.max(masked)
    i2 = jnp.min(jnp.where(masked == m2, iota, E))
    s = m1 + m2
    idx_ref[0] = i1
    idx_ref[1] = i2
    tkw_ref[0] = m1 / s
    tkw_ref[1] = m2 / s
    aux_ref[0] = E * m1
    for h in range(H):
        wh = inw_ref[h]
        xph = jax.lax.dot_general(
            x, wh, (((1,), (1,)), ((), ())),
            preferred_element_type=jnp.float32)
        xph = xph + inb_ref[h]
        mu = jnp.mean(xph, axis=0, keepdims=True)
        xc = xph - mu
        var = jnp.mean(xc * xc, axis=0, keepdims=True)
        xn_ref[h] = xc * jax.lax.rsqrt(var + 1e-5)


def _mixer(w, xn, b):
    e = jnp.exp(w)
    rinv = 1.0 / jnp.sum(e, axis=-1, keepdims=True)
    p = e * rinv
    acc = jnp.dot(p, xn,
                  preferred_element_type=jnp.float32)
    return acc + b


def _stage2_body(idx_ref, tkw_ref,
                 w0_ref, w1_ref, xn_ref, b0_ref, b1_ref,
                 owt_ref, ob_ref, y_ref, z_ref):
    h = pl.program_id(1)
    xn = xn_ref[h]
    mix0 = _mixer(w0_ref[0, 0], xn, b0_ref[0])
    mix1 = _mixer(w1_ref[0, 0], xn, b1_ref[0])
    z_ref[h] = tkw_ref[0] * mix0 + tkw_ref[1] * mix1

    @pl.when(h == H - 1)
    def _project():
        y = jnp.dot(z_ref[0], owt_ref[0],
                    preferred_element_type=jnp.float32)
        for h2 in range(1, H):
            y += jnp.dot(z_ref[h2], owt_ref[h2],
                         preferred_element_type=jnp.float32)
        y_ref[...] = y + ob_ref[...]


@jax.jit
def kernel(x, weight, bias, router_w, in_w, in_b, out_w, out_b):
    x2 = x.reshape(N, D)
    inw_h = in_w.reshape(H, HD, D)
    inb_h = in_b.reshape(H, 1, HD)

    xn, idx, tkw, aux = pl.pallas_call(
        _stage1_body,
        out_shape=(
            jax.ShapeDtypeStruct((H, N, HD), jnp.float32),
            jax.ShapeDtypeStruct((K,), jnp.int32),
            jax.ShapeDtypeStruct((K,), jnp.float32),
            jax.ShapeDtypeStruct((1,), jnp.float32),
        ),
        out_specs=(
            pl.BlockSpec(memory_space=pltpu.VMEM),
            pl.BlockSpec(memory_space=pltpu.SMEM),
            pl.BlockSpec(memory_space=pltpu.SMEM),
            pl.BlockSpec(memory_space=pltpu.SMEM),
        ),
        in_specs=[
            pl.BlockSpec(memory_space=pltpu.VMEM),
            pl.BlockSpec(memory_space=pltpu.VMEM),
            pl.BlockSpec(memory_space=pltpu.VMEM),
            pl.BlockSpec(memory_space=pltpu.VMEM),
        ],
    )(x2, inw_h, inb_h, router_w)

    bias3 = bias.reshape(E * H, N, 1)
    owt3 = out_w.T.reshape(H, HD, D)
    ob2 = out_b.reshape(1, D)

    grid_spec = pltpu.PrefetchScalarGridSpec(
        num_scalar_prefetch=2,
        grid=(NB, H),
        in_specs=[
            pl.BlockSpec((1, 1, R, N),
                         lambda nb, h, idx, tkw: (idx[0], h, nb, 0)),
            pl.BlockSpec((1, 1, R, N),
                         lambda nb, h, idx, tkw: (idx[1], h, nb, 0)),
            pl.BlockSpec((H, N, HD), lambda nb, h, idx, tkw: (0, 0, 0)),
            pl.BlockSpec((1, R, 1),
                         lambda nb, h, idx, tkw: (idx[0] * H + h, nb, 0)),
            pl.BlockSpec((1, R, 1),
                         lambda nb, h, idx, tkw: (idx[1] * H + h, nb, 0)),
            pl.BlockSpec((H, HD, D), lambda nb, h, idx, tkw: (0, 0, 0)),
            pl.BlockSpec((1, D), lambda nb, h, idx, tkw: (0, 0)),
        ],
        out_specs=pl.BlockSpec((R, D), lambda nb, h, idx, tkw: (nb, 0)),
        scratch_shapes=[pltpu.VMEM((H, R, HD), jnp.float32)],
    )

    y = pl.pallas_call(
        _stage2_body,
        grid_spec=grid_spec,
        out_shape=jax.ShapeDtypeStruct((N, D), jnp.float32),
    )(idx, tkw, weight, weight, xn, bias3, bias3, owt3, ob2)

    return y.reshape(B, N, D), aux.reshape(())


# speedup vs baseline: 1.0061x; 1.0061x over previous
import functools

import jax
import jax.numpy as jnp
from jax.experimental import pallas as pl
from jax.experimental.pallas import tpu as pltpu

B, N, D, E, H, K = 1, 2048, 768, 8, 8, 2
HD = D // H
R = 512
NB = N // R


def _stage1_body(x_ref, inw_ref, inb_ref, rw_ref,
                 xn_ref, idx_ref, tkw_ref, aux_ref):
    x = x_ref[...]
    rm = jnp.mean(x, axis=0, keepdims=True)
    logits = jax.lax.dot_general(
        rm, rw_ref[...], (((1,), (1,)), ((), ())),
        preferred_element_type=jnp.float32)
    lmax = jnp.max(logits)
    ex = jnp.exp(logits - lmax)
    probs = ex / jnp.sum(ex)
    iota = jax.lax.broadcasted_iota(jnp.int32, (1, E), 1)
    m1 = jnp.max(probs)
    i1 = jnp.min(jnp.where(probs == m1, iota, E))
    masked = jnp.where(iota == i1, -jnp.inf, probs)
    m2 = jnp.max(masked)
    i2 = jnp.min(jnp.where(masked == m2, iota, E))
    s = m1 + m2
    idx_ref[0] = i1
    idx_ref[1] = i2
    tkw_ref[0] = m1 / s
    tkw_ref[1] = m2 / s
    aux_ref[0] = E * m1
    for h in range(H):
        wh = inw_ref[h]
        xph = jax.lax.dot_general(
            x, wh, (((1,), (1,)), ((), ())),
            preferred_element_type=jnp.float32)
        xph = xph + inb_ref[h]
        mu = jnp.mean(xph, axis=0, keepdims=True)
        xc = xph - mu
        var = jnp.mean(xc * xc, axis=0, keepdims=True)
        xn_ref[h] = xc * jax.lax.rsqrt(var + 1e-5)


def _mixer(wl, wr, xn, b):
    el = jnp.exp(wl)
    er = jnp.exp(wr)
    s = (jnp.sum(el, axis=-1, keepdims=True)
         + jnp.sum(er, axis=-1, keepdims=True))
    rinv = 1.0 / s
    acc = (jnp.dot(el * rinv, xn[:N // 2],
                   preferred_element_type=jnp.float32)
           + jnp.dot(er * rinv, xn[N // 2:],
                     preferred_element_type=jnp.float32))
    return acc + b


def _stage2_body(idx_ref, tkw_ref,
                 w0l_ref, w0r_ref, w1l_ref, w1r_ref,
                 xn_ref, b0_ref, b1_ref,
                 owt_ref, ob_ref, y_ref, z_ref):
    h = pl.program_id(1)
    xn = xn_ref[h]
    mix0 = _mixer(w0l_ref[0, 0], w0r_ref[0, 0], xn, b0_ref[0])
    mix1 = _mixer(w1l_ref[0, 0], w1r_ref[0, 0], xn, b1_ref[0])
    z_ref[h] = tkw_ref[0] * mix0 + tkw_ref[1] * mix1

    @pl.when(h == H - 1)
    def _project():
        y = jnp.dot(z_ref[0], owt_ref[0],
                    preferred_element_type=jnp.float32)
        for h2 in range(1, H):
            y += jnp.dot(z_ref[h2], owt_ref[h2],
                         preferred_element_type=jnp.float32)
        y_ref[...] = y + ob_ref[...]


@jax.jit
def kernel(x, weight, bias, router_w, in_w, in_b, out_w, out_b):
    x2 = x.reshape(N, D)
    inw_h = in_w.reshape(H, HD, D)
    inb_h = in_b.reshape(H, 1, HD)

    xn, idx, tkw, aux = pl.pallas_call(
        _stage1_body,
        out_shape=(
            jax.ShapeDtypeStruct((H, N, HD), jnp.float32),
            jax.ShapeDtypeStruct((K,), jnp.int32),
            jax.ShapeDtypeStruct((K,), jnp.float32),
            jax.ShapeDtypeStruct((1,), jnp.float32),
        ),
        out_specs=(
            pl.BlockSpec(memory_space=pltpu.VMEM),
            pl.BlockSpec(memory_space=pltpu.SMEM),
            pl.BlockSpec(memory_space=pltpu.SMEM),
            pl.BlockSpec(memory_space=pltpu.SMEM),
        ),
        in_specs=[
            pl.BlockSpec(memory_space=pltpu.VMEM),
            pl.BlockSpec(memory_space=pltpu.VMEM),
            pl.BlockSpec(memory_space=pltpu.VMEM),
            pl.BlockSpec(memory_space=pltpu.VMEM),
        ],
    )(x2, inw_h, inb_h, router_w)

    bias3 = bias.reshape(E * H, N, 1)
    owt3 = out_w.T.reshape(H, HD, D)
    ob2 = out_b.reshape(1, D)

    grid_spec = pltpu.PrefetchScalarGridSpec(
        num_scalar_prefetch=2,
        grid=(NB, H),
        in_specs=[
            pl.BlockSpec((1, 1, R, N // 2),
                         lambda nb, h, idx, tkw: (idx[0], h, nb, 0)),
            pl.BlockSpec((1, 1, R, N // 2),
                         lambda nb, h, idx, tkw: (idx[0], h, nb, 1)),
            pl.BlockSpec((1, 1, R, N // 2),
                         lambda nb, h, idx, tkw: (idx[1], h, nb, 0)),
            pl.BlockSpec((1, 1, R, N // 2),
                         lambda nb, h, idx, tkw: (idx[1], h, nb, 1)),
            pl.BlockSpec((H, N, HD), lambda nb, h, idx, tkw: (0, 0, 0)),
            pl.BlockSpec((1, R, 1),
                         lambda nb, h, idx, tkw: (idx[0] * H + h, nb, 0)),
            pl.BlockSpec((1, R, 1),
                         lambda nb, h, idx, tkw: (idx[1] * H + h, nb, 0)),
            pl.BlockSpec((H, HD, D), lambda nb, h, idx, tkw: (0, 0, 0)),
            pl.BlockSpec((1, D), lambda nb, h, idx, tkw: (0, 0)),
        ],
        out_specs=pl.BlockSpec((R, D), lambda nb, h, idx, tkw: (nb, 0)),
        scratch_shapes=[pltpu.VMEM((H, R, HD), jnp.float32)],
    )

    y = pl.pallas_call(
        _stage2_body,
        grid_spec=grid_spec,
        out_shape=jax.ShapeDtypeStruct((N, D), jnp.float32),
    )(idx, tkw, weight, weight, weight, weight, xn, bias3, bias3, owt3, ob2)

    return y.reshape(B, N, D), aux.reshape(())


# speedup vs baseline: 1.0204x; 1.0142x over previous
import functools

import jax
import jax.numpy as jnp
from jax.experimental import pallas as pl
from jax.experimental.pallas import tpu as pltpu

B, N, D, E, H, K = 1, 2048, 768, 8, 8, 2
HD = D // H
R = 512
NB = N // R


def _stage1_body(x_ref, inw_ref, inb_ref, rw_ref, ow_ref,
                 xn_ref, idx_ref, tkw_ref, aux_ref, owt_ref):
    x = x_ref[...]
    rm = jnp.mean(x, axis=0, keepdims=True)
    logits = jax.lax.dot_general(
        rm, rw_ref[...], (((1,), (1,)), ((), ())),
        preferred_element_type=jnp.float32)
    lmax = jnp.max(logits)
    ex = jnp.exp(logits - lmax)
    probs = ex / jnp.sum(ex)
    iota = jax.lax.broadcasted_iota(jnp.int32, (1, E), 1)
    m1 = jnp.max(probs)
    i1 = jnp.min(jnp.where(probs == m1, iota, E))
    masked = jnp.where(iota == i1, -jnp.inf, probs)
    m2 = jnp.max(masked)
    i2 = jnp.min(jnp.where(masked == m2, iota, E))
    s = m1 + m2
    idx_ref[0] = i1
    idx_ref[1] = i2
    tkw_ref[0] = m1 / s
    tkw_ref[1] = m2 / s
    aux_ref[0] = E * m1
    for h in range(H):
        wh = inw_ref[h]
        xph = jax.lax.dot_general(
            x, wh, (((1,), (1,)), ((), ())),
            preferred_element_type=jnp.float32)
        xph = xph + inb_ref[h]
        mu = jnp.mean(xph, axis=0, keepdims=True)
        xc = xph - mu
        var = jnp.mean(xc * xc, axis=0, keepdims=True)
        xn_ref[h] = xc * jax.lax.rsqrt(var + 1e-5)
        owt_ref[h] = ow_ref[:, h * HD:(h + 1) * HD].T


def _mixer(wl, wr, xn, b):
    el = jnp.exp(wl)
    er = jnp.exp(wr)
    s = (jnp.sum(el, axis=-1, keepdims=True)
         + jnp.sum(er, axis=-1, keepdims=True))
    rinv = 1.0 / s
    acc = (jnp.dot(el * rinv, xn[:N // 2],
                   preferred_element_type=jnp.float32)
           + jnp.dot(er * rinv, xn[N // 2:],
                     preferred_element_type=jnp.float32))
    return acc + b


def _stage2_body(idx_ref, tkw_ref,
                 w0l_ref, w0r_ref, w1l_ref, w1r_ref,
                 xn_ref, b0_ref, b1_ref,
                 owt_ref, ob_ref, y_ref, z_ref):
    h = pl.program_id(1)
    xn = xn_ref[h]
    mix0 = _mixer(w0l_ref[0, 0], w0r_ref[0, 0], xn, b0_ref[0])
    mix1 = _mixer(w1l_ref[0, 0], w1r_ref[0, 0], xn, b1_ref[0])
    z_ref[h] = tkw_ref[0] * mix0 + tkw_ref[1] * mix1

    @pl.when(h == H - 1)
    def _project():
        y = jnp.dot(z_ref[0], owt_ref[0],
                    preferred_element_type=jnp.float32)
        for h2 in range(1, H):
            y += jnp.dot(z_ref[h2], owt_ref[h2],
                         preferred_element_type=jnp.float32)
        y_ref[...] = y + ob_ref[...]


@jax.jit
def kernel(x, weight, bias, router_w, in_w, in_b, out_w, out_b):
    x2 = x.reshape(N, D)
    inw_h = in_w.reshape(H, HD, D)
    inb_h = in_b.reshape(H, 1, HD)

    xn, idx, tkw, aux, owt3 = pl.pallas_call(
        _stage1_body,
        out_shape=(
            jax.ShapeDtypeStruct((H, N, HD), jnp.float32),
            jax.ShapeDtypeStruct((K,), jnp.int32),
            jax.ShapeDtypeStruct((K,), jnp.float32),
            jax.ShapeDtypeStruct((1,), jnp.float32),
            jax.ShapeDtypeStruct((H, HD, D), jnp.float32),
        ),
        out_specs=(
            pl.BlockSpec(memory_space=pltpu.VMEM),
            pl.BlockSpec(memory_space=pltpu.SMEM),
            pl.BlockSpec(memory_space=pltpu.SMEM),
            pl.BlockSpec(memory_space=pltpu.SMEM),
            pl.BlockSpec(memory_space=pltpu.VMEM),
        ),
        in_specs=[
            pl.BlockSpec(memory_space=pltpu.VMEM),
            pl.BlockSpec(memory_space=pltpu.VMEM),
            pl.BlockSpec(memory_space=pltpu.VMEM),
            pl.BlockSpec(memory_space=pltpu.VMEM),
            pl.BlockSpec(memory_space=pltpu.VMEM),
        ],
    )(x2, inw_h, inb_h, router_w, out_w)

    bias3 = bias.reshape(E * H, N, 1)
    ob2 = out_b.reshape(1, D)

    grid_spec = pltpu.PrefetchScalarGridSpec(
        num_scalar_prefetch=2,
        grid=(NB, H),
        in_specs=[
            pl.BlockSpec((1, 1, R, N // 2),
                         lambda nb, h, idx, tkw: (idx[0], h, nb, 0)),
            pl.BlockSpec((1, 1, R, N // 2),
                         lambda nb, h, idx, tkw: (idx[0], h, nb, 1)),
            pl.BlockSpec((1, 1, R, N // 2),
                         lambda nb, h, idx, tkw: (idx[1], h, nb, 0)),
            pl.BlockSpec((1, 1, R, N // 2),
                         lambda nb, h, idx, tkw: (idx[1], h, nb, 1)),
            pl.BlockSpec((H, N, HD), lambda nb, h, idx, tkw: (0, 0, 0)),
            pl.BlockSpec((1, R, 1),
                         lambda nb, h, idx, tkw: (idx[0] * H + h, nb, 0)),
            pl.BlockSpec((1, R, 1),
                         lambda nb, h, idx, tkw: (idx[1] * H + h, nb, 0)),
            pl.BlockSpec((H, HD, D), lambda nb, h, idx, tkw: (0, 0, 0)),
            pl.BlockSpec((1, D), lambda nb, h, idx, tkw: (0, 0)),
        ],
        out_specs=pl.BlockSpec((R, D), lambda nb, h, idx, tkw: (nb, 0)),
        scratch_shapes=[pltpu.VMEM((H, R, HD), jnp.float32)],
    )

    y = pl.pallas_call(
        _stage2_body,
        grid_spec=grid_spec,
        out_shape=jax.ShapeDtypeStruct((N, D), jnp.float32),
    )(idx, tkw, weight, weight, weight, weight, xn, bias3, bias3, owt3, ob2)

    return y.reshape(B, N, D), aux.reshape(())
